# baseline (speedup 1.0000x reference)
import functools

import jax
import jax.numpy as jnp
from jax import lax
from jax.experimental import pallas as pl
from jax.experimental.pallas import tpu as pltpu
from jax.experimental.pallas import tpu_sc as plsc

_BATCH = 16384
_D = 32
_H1 = 64
_H2 = 32
_NCLS = 2
_NC = 2
_NS = 16
_NW = _NC * _NS
_ROWS = 1000000
_CH = 256
_NCID = (_ROWS + _CH - 1) // _CH
_KMAX = (_NCID + _NW - 1) // _NW
_RING = 8
_RNDS = (_KMAX + _RING - 1) // _RING
_PHYS_LANES = 1000064
_LAST_START = _PHYS_LANES - _CH
_SROWS = 128
_LISTN = _BATCH + 16

_sc_mesh = plsc.VectorSubcoreMesh(core_axis_name="c", subcore_axis_name="s")


@functools.partial(
    pl.kernel,
    mesh=_sc_mesh,
    out_type=(
        jax.ShapeDtypeStruct((_BATCH + 1, 128), jnp.float32),
        jax.ShapeDtypeStruct((_BATCH + 1, 128), jnp.float32),
    ),
    scratch_types=[
        pltpu.VMEM((1024,), jnp.int32),
        pltpu.VMEM((_LISTN,), jnp.int32),
        pltpu.VMEM((_LISTN,), jnp.int32),
        pltpu.VMEM((_RING, _D, _CH), jnp.float32),
        pltpu.VMEM((_SROWS, 128), jnp.float32),
        pltpu.VMEM((_SROWS,), jnp.int32),
        pltpu.SemaphoreType.DMA,
        pltpu.SemaphoreType.DMA,
        pltpu.SemaphoreType.DMA,
    ],
    compiler_params=pltpu.CompilerParams(use_tc_tiling_on_sc=True,
                                         needs_layout_passes=False),
)
def _sc_gather(ut_t, it_t, uidx, iidx, ue_out, ie_out,
               ipiece, plist, mlist, bufs, staging, sidx,
               ssem, fsem, csem):
    w = lax.axis_index("s") * _NC + lax.axis_index("c")
    iota = jnp.arange(16, dtype=jnp.int32)
    trash = jnp.full((16,), _BATCH, dtype=jnp.int32)

    def _pc(m):
        c = plsc.all_reduce_population_count(m)
        return c[0] if getattr(c, "ndim", 0) else c

    def _reset_sidx():
        for t in range(_SROWS // 16):
            sidx[pl.ds(t * 16, 16)] = trash

    _reset_sidx()

    def _run_table(tab, idx_hbm, out_hbm):
        def _piece(p, cnt):
            cp = pltpu.async_copy(idx_hbm.at[pl.ds(p * 1024, 1024)], ipiece, ssem)
            cp.wait()

            def _seg(s, cnt):
                rv = ipiece[pl.ds(s * 16, 16)]
                cidv = rv >> 8
                m = (cidv & (_NW - 1)) == w
                startv = jnp.minimum(cidv << 8, _LAST_START)
                offv = rv - startv
                bpv = (p * 1024 + s * 16) + iota
                key = ((cidv >> 5) << 22) | (bpv << 8) | offv
                plsc.store_compressed(plist.at[pl.ds(cnt, 16)], key, mask=m)
                return cnt + _pc(m)

            return lax.fori_loop(0, 64, _seg, cnt)

        cnt = lax.fori_loop(0, 16, _piece, jnp.int32(0))
        nseg = (cnt + 15) >> 4

        def _cid(k):
            return k * _NW + w

        def _start_of(cid):
            return pl.multiple_of(jnp.minimum(cid * _CH, _LAST_START), 128)

        def _fire(k, buf):
            @pl.when(_cid(k) < _NCID)
            def _():
                pltpu.make_async_copy(
                    tab.at[:, pl.ds(_start_of(_cid(k)), _CH)], buf, fsem
                ).start()

        def _wait_fetch(k, buf):
            @pl.when(_cid(k) < _NCID)
            def _():
                pltpu.make_async_copy(
                    tab.at[:, pl.ds(0, _CH)], buf, fsem
                ).wait()

        def _flush():
            pltpu.async_copy(staging, out_hbm.at[sidx], csem).wait()
            _reset_sidx()
            return jnp.int32(0)

        def _process(k, buf, fill0):
            def _seg2(s, cc):
                kv = plist[pl.ds(s * 16, 16)]
                pos = s * 16 + iota
                m = (pos < cnt) & ((kv >> 22) == k)
                plsc.store_compressed(mlist.at[pl.ds(cc, 16)], kv, mask=m)
                return cc + _pc(m)

            cc = lax.fori_loop(0, nseg, _seg2, jnp.int32(0))
            ngrp = (cc + 15) >> 4

            def _grp(p, fill):
                kv = mlist[pl.ds(p * 16, 16)]
                offs = kv & (_CH - 1)
                bps = (kv >> 8) & (_BATCH - 1)
                valid = iota < (cc - p * 16)
                rows = fill + iota
                for d in range(_D):
                    dv = jnp.full((16,), d, dtype=jnp.int32)
                    v = plsc.load_gather(buf, [dv, offs])
                    plsc.store_scatter(staging, [rows, dv], v, mask=valid)
                plsc.store_scatter(sidx, [rows], bps, mask=valid)
                fill = fill + _pc(valid)
                return lax.cond(fill > _SROWS - 16, _flush, lambda: fill)

            return lax.fori_loop(0, ngrp, _grp, fill0)

        for sl in range(_RING):
            _fire(sl, bufs.at[sl])

        def _round(r, fill):
            for sl in range(_RING):
                k = r * _RING + sl
                _wait_fetch(k, bufs.at[sl])
                fill = _process(k, bufs.at[sl], fill)
                _fire(k + _RING, bufs.at[sl])
            return fill

        fill = lax.fori_loop(0, _RNDS, _round, jnp.int32(0))
        lax.cond(fill > 0, _flush, lambda: fill)

    _run_table(ut_t, uidx, ue_out)
    _run_table(it_t, iidx, ie_out)


_BM = 2048
_GRID = _BATCH // _BM


def _mlp_body(ue, ie, w1u, w1i, b1, w2, b2, w3, b3, out):
    x = jnp.dot(ue[:, :_D], w1u[...], preferred_element_type=jnp.float32)
    x = x + jnp.dot(ie[:, :_D], w1i[...], preferred_element_type=jnp.float32)
    x = jnp.maximum(x + b1[...], 0.0)
    x = jnp.maximum(jnp.dot(x, w2[...], preferred_element_type=jnp.float32) + b2[...], 0.0)
    x = jnp.maximum(jnp.dot(x, w3[...], preferred_element_type=jnp.float32) + b3[...], 0.0)
    out[...] = x


_mlp = pl.pallas_call(
    _mlp_body,
    grid=(_GRID,),
    in_specs=[
        pl.BlockSpec((_BM, 128), lambda i: (i, 0)),
        pl.BlockSpec((_BM, 128), lambda i: (i, 0)),
        pl.BlockSpec((_D, _H1), lambda i: (0, 0)),
        pl.BlockSpec((_D, _H1), lambda i: (0, 0)),
        pl.BlockSpec((1, _H1), lambda i: (0, 0)),
        pl.BlockSpec((_H1, _H2), lambda i: (0, 0)),
        pl.BlockSpec((1, _H2), lambda i: (0, 0)),
        pl.BlockSpec((_H2, _NCLS), lambda i: (0, 0)),
        pl.BlockSpec((1, _NCLS), lambda i: (0, 0)),
    ],
    out_specs=pl.BlockSpec((_BM, _NCLS), lambda i: (i, 0)),
    out_shape=jax.ShapeDtypeStruct((_BATCH, _NCLS), jnp.float32),
)


def kernel(user_input, item_input, user_table, item_table, W1, b1, W2, b2, W3, b3):
    ue, ie = _sc_gather(user_table.T, item_table.T,
                        user_input.astype(jnp.int32),
                        item_input.astype(jnp.int32))
    return _mlp(ue, ie, W1[:_D], W1[_D:], b1.reshape(1, _H1),
                W2, b2.reshape(1, _H2), W3, b3.reshape(1, _NCLS))


# speedup vs baseline: 2.8149x; 2.8149x over previous
import functools

import jax
import jax.numpy as jnp
from jax import lax
from jax.experimental import pallas as pl
from jax.experimental.pallas import tpu as pltpu
from jax.experimental.pallas import tpu_sc as plsc

_BATCH = 16384
_D = 32
_H1 = 64
_H2 = 32
_NCLS = 2
_NC = 2
_NS = 16
_NW = _NC * _NS
_BPW = _BATCH // _NW
_G = 8
_NGRP = _BPW // _G
_LANES = 128

_sc_mesh = plsc.VectorSubcoreMesh(core_axis_name="c", subcore_axis_name="s")


@functools.partial(
    pl.kernel,
    mesh=_sc_mesh,
    out_type=(
        jax.ShapeDtypeStruct((_D, _BATCH), jnp.float32),
        jax.ShapeDtypeStruct((_D, _BATCH), jnp.float32),
    ),
    scratch_types=[
        pltpu.VMEM((_BPW + 16,), jnp.int32),
        pltpu.VMEM((2 * _G, _D, _LANES), jnp.float32),
        pltpu.VMEM((_D, _BPW), jnp.float32),
        pltpu.VMEM((_D, _BPW), jnp.float32),
        pltpu.SemaphoreType.DMA,
        pltpu.SemaphoreType.DMA,
    ],
    compiler_params=pltpu.CompilerParams(use_tc_tiling_on_sc=True,
                                         needs_layout_passes=False),
)
def _sc_gather(ut_t, it_t, uidx, iidx, ue_out, ie_out,
               idx_v, blocks, ue_v, ie_v, fsem, ssem):
    wid = lax.axis_index("s") * _NC + lax.axis_index("c")
    base = wid * _BPW
    rows_lo = jnp.arange(16, dtype=jnp.int32)
    rows_hi = rows_lo + 16

    def _fire_one(tab, r, slot):
        start = pl.multiple_of((r >> 7) * _LANES, _LANES)
        pltpu.make_async_copy(
            tab.at[:, pl.ds(start, _LANES)], blocks.at[slot], fsem
        ).start()

    def _extract_one(tab, out_v, r, j, slot):
        pltpu.make_async_copy(
            tab.at[:, pl.ds(0, _LANES)], blocks.at[slot], fsem
        ).wait()
        lvec = jnp.full((16,), r & 127, dtype=jnp.int32)
        jvec = jnp.full((16,), j, dtype=jnp.int32)
        blk = blocks.at[slot]
        v0 = plsc.load_gather(blk, [rows_lo, lvec])
        v1 = plsc.load_gather(blk, [rows_hi, lvec])
        plsc.store_scatter(out_v, [rows_lo, jvec], v0)
        plsc.store_scatter(out_v, [rows_hi, jvec], v1)

    def _run_table(tab, idx_hbm, out_v, out_hbm):
        dst = idx_v.at[pl.ds(0, _BPW)]
        pltpu.make_async_copy(idx_hbm.at[pl.ds(base, _BPW)], dst, ssem).start()
        pltpu.make_async_copy(idx_hbm.at[pl.ds(base, _BPW)], dst, ssem).wait()

        def _fire_group(g, slot0):
            iv = idx_v[pl.ds(g * _G, 16)]
            for kk in range(_G):
                _fire_one(tab, iv[kk], slot0 + kk)

        def _extract_group(g, slot0):
            iv = idx_v[pl.ds(g * _G, 16)]
            for kk in range(_G):
                _extract_one(tab, out_v, iv[kk], g * _G + kk, slot0 + kk)

        _fire_group(0, 0)

        def _body(h, carry):
            g = 2 * h
            _fire_group(g + 1, _G)
            _extract_group(g, 0)

            @pl.when(h < _NGRP // 2 - 1)
            def _():
                _fire_group(g + 2, 0)

            _extract_group(g + 1, _G)
            return carry

        lax.fori_loop(0, _NGRP // 2, _body, 0)
        pltpu.sync_copy(out_v, out_hbm.at[:, pl.ds(base, _BPW)])

    _run_table(ut_t, uidx, ue_v, ue_out)
    _run_table(it_t, iidx, ie_v, ie_out)


_BM = 16384
_GRID = _BATCH // _BM


def _mlp_body(ue, ie, w1u, w1i, b1, w2, b2, w3, b3, out):
    x = jnp.dot(w1u[...], ue[...], preferred_element_type=jnp.float32)
    x = x + jnp.dot(w1i[...], ie[...], preferred_element_type=jnp.float32)
    x = jnp.maximum(x + b1[...], 0.0)
    x = jnp.maximum(jnp.dot(w2[...], x, preferred_element_type=jnp.float32) + b2[...], 0.0)
    x = jnp.maximum(jnp.dot(w3[...], x, preferred_element_type=jnp.float32) + b3[...], 0.0)
    out[...] = x


_mlp = pl.pallas_call(
    _mlp_body,
    grid=(_GRID,),
    in_specs=[
        pl.BlockSpec((_D, _BM), lambda i: (0, i)),
        pl.BlockSpec((_D, _BM), lambda i: (0, i)),
        pl.BlockSpec((_H1, _D), lambda i: (0, 0)),
        pl.BlockSpec((_H1, _D), lambda i: (0, 0)),
        pl.BlockSpec((_H1, 1), lambda i: (0, 0)),
        pl.BlockSpec((_H2, _H1), lambda i: (0, 0)),
        pl.BlockSpec((_H2, 1), lambda i: (0, 0)),
        pl.BlockSpec((_NCLS, _H2), lambda i: (0, 0)),
        pl.BlockSpec((_NCLS, 1), lambda i: (0, 0)),
    ],
    out_specs=pl.BlockSpec((_NCLS, _BM), lambda i: (0, i)),
    out_shape=jax.ShapeDtypeStruct((_NCLS, _BATCH), jnp.float32),
)


def kernel(user_input, item_input, user_table, item_table, W1, b1, W2, b2, W3, b3):
    ue_t, ie_t = _sc_gather(user_table.T, item_table.T,
                            user_input.astype(jnp.int32),
                            item_input.astype(jnp.int32))
    w1t = W1.T
    out_t = _mlp(ue_t, ie_t, w1t[:, :_D], w1t[:, _D:], b1.reshape(_H1, 1),
                 W2.T, b2.reshape(_H2, 1), W3.T, b3.reshape(_NCLS, 1))
    return out_t.T
